# Initial kernel scaffold; baseline (speedup 1.0000x reference)
import jax
import jax.numpy as jnp
from jax.experimental import pallas as pl


def kernel(x, edge_index, batch, W1, b1, W2, b2, W3, b3, W4, b4, fcW1, fcb1, fcW2, fcb2):
    raise NotImplementedError("write your pallas kernel here")


# speedup vs baseline: 9.7633x; 9.7633x over previous
import functools

import jax
import jax.numpy as jnp
from jax import lax
from jax.experimental import pallas as pl
from jax.experimental.pallas import tpu as pltpu
from jax.experimental.pallas import tpu_sc as plsc

N = 50000
G = 64
NC = 2
NS = 16
NW = NC * NS
NP = 51200
SLICE = NP // NS
IDXC = 40
TN = 512
W1R = 8


def _sc_agg(g_list, src2d, dst2d, zrow, wr):
  C = len(g_list)
  epr = src2d.shape[0]
  rpt = epr // NW
  idxc = IDXC
  mesh = plsc.VectorSubcoreMesh(core_axis_name="c", subcore_axis_name="s")
  out_type = [jax.ShapeDtypeStruct((NC, NP, wr), jnp.float32) for _ in range(C)]
  scratch = [
      pltpu.VMEM((idxc, 128), jnp.int32),
      pltpu.VMEM((idxc, 128), jnp.int32),
      pltpu.VMEM((128, wr), jnp.float32),
      pltpu.VMEM((128, wr), jnp.float32),
      pltpu.VMEM((128, wr), jnp.float32),
      pltpu.VMEM_SHARED((NP, wr), jnp.float32),
      pltpu.SemaphoreType.DMA,
      pltpu.SemaphoreType.DMA,
  ]

  @functools.partial(pl.kernel, mesh=mesh, out_type=out_type,
                     scratch_types=scratch,
                     compiler_params=pltpu.CompilerParams(
                         use_tc_tiling_on_sc=False))
  def k(*refs):
    gs = refs[:C]
    src_hbm, dst_hbm, z_hbm = refs[C], refs[C + 1], refs[C + 2]
    outs = refs[C + 3:C + 3 + C]
    src_v, dst_v, buf0, buf1, zbuf, acc, sem0, sem1 = refs[C + 3 + C:]

    cid = lax.axis_index("c")
    sid = lax.axis_index("s")
    wid = sid * NC + cid
    row0 = wid * rpt
    my_lo = sid * SLICE

    pltpu.sync_copy(z_hbm, zbuf)

    for c in range(C):
      g_hbm = gs[c]
      for z in range(SLICE // 128):
        pltpu.sync_copy(zbuf, acc.at[pl.ds(my_lo + z * 128, 128)])
      plsc.subcore_barrier()

      for st in range(rpt // idxc):
        pltpu.sync_copy(src_hbm.at[pl.ds(row0 + st * idxc, idxc)], src_v)
        pltpu.sync_copy(dst_hbm.at[pl.ds(row0 + st * idxc, idxc)], dst_v)
        pltpu.async_copy(g_hbm.at[src_v.at[0]], buf0, sem0)

        def body(it, _):
          j = it * 2
          pltpu.make_async_copy(g_hbm.at[src_v.at[j]], buf0, sem0).wait()
          pltpu.async_copy(g_hbm.at[src_v.at[j + 1]], buf1, sem1)
          pltpu.sync_copy(buf0, acc.at[dst_v.at[j]], add=True)
          pltpu.make_async_copy(g_hbm.at[src_v.at[j + 1]], buf1, sem1).wait()

          @pl.when(j + 2 < idxc)
          def _():
            pltpu.async_copy(g_hbm.at[src_v.at[j + 2]], buf0, sem0)

          pltpu.sync_copy(buf1, acc.at[dst_v.at[j + 1]], add=True)
          return 0

        lax.fori_loop(0, idxc // 2, body, 0)
      plsc.subcore_barrier()

      pltpu.sync_copy(acc.at[pl.ds(my_lo, SLICE)],
                      outs[c].at[cid, pl.ds(my_lo, SLICE)])
      plsc.subcore_barrier()

  res = k(*g_list, src2d, dst2d, zrow)
  return list(res) if isinstance(res, (tuple, list)) else [res]


def _sc_degree(dst2d, ones_row, zrow):
  epr = dst2d.shape[0]
  rpt = epr // NW
  wr = ones_row.shape[1]
  mesh = plsc.VectorSubcoreMesh(core_axis_name="c", subcore_axis_name="s")
  scratch = [
      pltpu.VMEM((rpt, 128), jnp.int32),
      pltpu.VMEM((128, wr), jnp.float32),
      pltpu.VMEM((128, wr), jnp.float32),
      pltpu.VMEM_SHARED((NP, wr), jnp.float32),
  ]

  @functools.partial(
      pl.kernel, mesh=mesh,
      out_type=jax.ShapeDtypeStruct((NC, NP, wr), jnp.float32),
      scratch_types=scratch,
      compiler_params=pltpu.CompilerParams(use_tc_tiling_on_sc=False))
  def k(dst_hbm, ones_hbm, z_hbm, out_hbm, dst_v, obuf, zbuf, acc):
    cid = lax.axis_index("c")
    sid = lax.axis_index("s")
    wid = sid * NC + cid
    row0 = wid * rpt
    my_lo = sid * SLICE

    pltpu.sync_copy(dst_hbm.at[pl.ds(row0, rpt)], dst_v)
    pltpu.sync_copy(ones_hbm, obuf)
    pltpu.sync_copy(z_hbm, zbuf)
    for z in range(SLICE // 128):
      pltpu.sync_copy(zbuf, acc.at[pl.ds(my_lo + z * 128, 128)])
    plsc.subcore_barrier()

    def body(j, _):
      pltpu.sync_copy(obuf, acc.at[dst_v.at[j]], add=True)
      return 0

    lax.fori_loop(0, rpt, body, 0)
    plsc.subcore_barrier()
    pltpu.sync_copy(acc.at[pl.ds(my_lo, SLICE)],
                    out_hbm.at[cid, pl.ds(my_lo, SLICE)])

  return k(dst2d, ones_row, zrow)


def _tc_prep(deg_p, x_pad):
  grid = (NP // TN,)

  def body(deg_ref, x_ref, a_ref, g_ref):
    i = pl.program_id(0)
    rows = i * TN + lax.broadcasted_iota(jnp.int32, (TN, 1), 0)
    d = deg_ref[0][:, :1] + deg_ref[1][:, :1] + 1.0
    a = jnp.where(rows < N, lax.rsqrt(d), 0.0)
    a_ref[...] = a
    col0 = lax.broadcasted_iota(jnp.int32, (TN, W1R), 1) == 0
    g_ref[...] = jnp.where(col0, a * x_ref[...], 0.0)

  return pl.pallas_call(
      body, grid=grid,
      in_specs=[pl.BlockSpec((NC, TN, W1R), lambda i: (0, i, 0)),
                pl.BlockSpec((TN, 1), lambda i: (i, 0))],
      out_specs=[pl.BlockSpec((TN, 1), lambda i: (i, 0)),
                 pl.BlockSpec((TN, W1R), lambda i: (i, 0))],
      out_shape=[jax.ShapeDtypeStruct((NP, 1), jnp.float32),
                 jax.ShapeDtypeStruct((NP, W1R), jnp.float32)],
  )(deg_p, x_pad)


def _tc_layer1(P, g1, a, W1, b1):
  grid = (NP // TN,)

  def body(p_ref, g_ref, a_ref, w_ref, b_ref, o_ref):
    av = a_ref[...]
    agg = av * (p_ref[0][:, :1] + p_ref[1][:, :1] + g_ref[:, :1])
    h = jnp.maximum(agg * w_ref[...] + b_ref[...], 0.0)
    o_ref[...] = av * h

  return pl.pallas_call(
      body, grid=grid,
      in_specs=[pl.BlockSpec((NC, TN, W1R), lambda i: (0, i, 0)),
                pl.BlockSpec((TN, W1R), lambda i: (i, 0)),
                pl.BlockSpec((TN, 1), lambda i: (i, 0)),
                pl.BlockSpec((1, 32), lambda i: (0, 0)),
                pl.BlockSpec((1, 32), lambda i: (0, 0))],
      out_specs=pl.BlockSpec((TN, 32), lambda i: (i, 0)),
      out_shape=jax.ShapeDtypeStruct((NP, 32), jnp.float32),
  )(P, g1, a, W1, b1)


def _tc_layer(P_list, g_list, a, W, b, c_out, g_out):
  C = len(g_list)
  f_out = W.shape[1]
  grid = (NP // TN,)

  def body(*refs):
    p_refs = refs[:C]
    g_refs = refs[C:2 * C]
    a_ref, w_ref, b_ref = refs[2 * C], refs[2 * C + 1], refs[2 * C + 2]
    outs = refs[2 * C + 3:]
    av = a_ref[...]
    acc = None
    for c in range(C):
      pc = p_refs[c]
      aggc = av * (pc[0] + pc[1] + g_refs[c][...])
      part = jnp.dot(aggc, w_ref[c * 32:(c + 1) * 32, :],
                     preferred_element_type=jnp.float32)
      acc = part if acc is None else acc + part
    h = jnp.maximum(acc + b_ref[...], 0.0)
    if g_out:
      gh = av * h
      for c2 in range(len(outs)):
        outs[c2][...] = gh[:, c2 * 32:(c2 + 1) * 32]
    else:
      outs[0][...] = h

  in_specs = ([pl.BlockSpec((NC, TN, 32), lambda i: (0, i, 0))] * C +
              [pl.BlockSpec((TN, 32), lambda i: (i, 0))] * C +
              [pl.BlockSpec((TN, 1), lambda i: (i, 0)),
               pl.BlockSpec(W.shape, lambda i: (0, 0)),
               pl.BlockSpec((1, f_out), lambda i: (0, 0))])
  if g_out:
    out_specs = [pl.BlockSpec((TN, 32), lambda i: (i, 0))] * c_out
    out_shape = [jax.ShapeDtypeStruct((NP, 32), jnp.float32)] * c_out
  else:
    out_specs = [pl.BlockSpec((TN, f_out), lambda i: (i, 0))]
    out_shape = [jax.ShapeDtypeStruct((NP, f_out), jnp.float32)]
  res = pl.pallas_call(body, grid=grid, in_specs=in_specs,
                       out_specs=out_specs, out_shape=out_shape)(
                           *P_list, *g_list, a, W, b)
  return res


def _tc_pool_mlp(h5, batch3d, fcW1, fcb1, fcW2, fcb2):
  grid = (NP // TN,)
  nb = grid[0]

  def body(h_ref, b_ref, w1_ref, c1_ref, w2_ref, c2_ref, o_ref, s_ref, n_ref):
    i = pl.program_id(0)

    @pl.when(i == 0)
    def _():
      s_ref[...] = jnp.zeros_like(s_ref)
      n_ref[...] = jnp.zeros_like(n_ref)

    ids = b_ref[0]
    gids = lax.broadcasted_iota(jnp.int32, (G, TN), 0)
    mask = (gids == ids).astype(jnp.float32)
    s_ref[...] += jnp.dot(mask, h_ref[...], preferred_element_type=jnp.float32)
    n_ref[...] += jnp.sum(mask, axis=1, keepdims=True)

    @pl.when(i == nb - 1)
    def _():
      mean = s_ref[...] / jnp.clip(n_ref[...], 1.0, None)
      r = jnp.maximum(jnp.dot(mean, w1_ref[...],
                              preferred_element_type=jnp.float32) +
                      c1_ref[...], 0.0)
      o_ref[...] = jnp.dot(r, w2_ref[...],
                           preferred_element_type=jnp.float32) + c2_ref[...]

  return pl.pallas_call(
      body, grid=grid,
      in_specs=[pl.BlockSpec((TN, 256), lambda i: (i, 0)),
                pl.BlockSpec((1, 1, TN), lambda i: (i, 0, 0)),
                pl.BlockSpec((256, 128), lambda i: (0, 0)),
                pl.BlockSpec((1, 128), lambda i: (0, 0)),
                pl.BlockSpec((128, 10), lambda i: (0, 0)),
                pl.BlockSpec((1, 10), lambda i: (0, 0))],
      out_specs=pl.BlockSpec((G, 10), lambda i: (0, 0)),
      out_shape=jax.ShapeDtypeStruct((G, 10), jnp.float32),
      scratch_shapes=[pltpu.VMEM((G, 256), jnp.float32),
                      pltpu.VMEM((G, 1), jnp.float32)],
  )(h5, batch3d, fcW1, fcb1, fcW2, fcb2)


def kernel(x, edge_index, batch, W1, b1, W2, b2, W3, b3, W4, b4,
           fcW1, fcb1, fcW2, fcb2):
  E = edge_index.shape[1]
  rpt = (E + NW * 128 - 1) // (NW * 128)
  rpt = ((rpt + IDXC - 1) // IDXC) * IDXC
  epr = rpt * NW
  ep = epr * 128
  pad = ep - E
  padv = jnp.full((pad,), N, dtype=jnp.int32)
  src2d = jnp.concatenate([edge_index[0], padv]).reshape(epr, 128)
  dst2d = jnp.concatenate([edge_index[1], padv]).reshape(epr, 128)

  x_pad = jnp.pad(x, ((0, NP - N), (0, 0)))
  batch3d = jnp.pad(batch, (0, NP - N), constant_values=G).reshape(
      NP // TN, 1, TN)

  z1 = jnp.zeros((128, W1R), jnp.float32)
  o1 = jnp.ones((128, W1R), jnp.float32)
  z32 = jnp.zeros((128, 32), jnp.float32)

  deg_p = _sc_degree(dst2d, o1, z1)
  a, g1 = _tc_prep(deg_p, x_pad)

  (P1,) = _sc_agg([g1], src2d, dst2d, z1, W1R)
  g2 = _tc_layer1(P1, g1, a, W1.astype(jnp.float32),
                  b1.reshape(1, 32).astype(jnp.float32))

  (P2,) = _sc_agg([g2], src2d, dst2d, z32, 32)
  g3 = _tc_layer([P2], [g2], a, W2, b2.reshape(1, 64), 2, True)

  P3 = _sc_agg(list(g3), src2d, dst2d, z32, 32)
  g4 = _tc_layer(P3, list(g3), a, W3, b3.reshape(1, 128), 4, True)

  P4 = _sc_agg(list(g4), src2d, dst2d, z32, 32)
  (h5,) = _tc_layer(P4, list(g4), a, W4, b4.reshape(1, 256), 0, False)

  return _tc_pool_mlp(h5, batch3d, fcW1, fcb1.reshape(1, 128),
                      fcW2, fcb2.reshape(1, 10))


# speedup vs baseline: 14.1793x; 1.4523x over previous
import functools

import jax
import jax.numpy as jnp
from jax import lax
from jax.experimental import pallas as pl
from jax.experimental.pallas import tpu as pltpu
from jax.experimental.pallas import tpu_sc as plsc

N = 50000
G = 64
NC = 2
NS = 16
NW = NC * NS
NP = 51200
SLICE = NP // NS
CW = 16
IDXC = 40
TN = 512
W1R = 8


def _sc_agg(g_list, src2d, dst2d, zrow, wr):
  C = len(g_list)
  epr = src2d.shape[0]
  rpt = epr // NW
  idxc = IDXC
  mesh = plsc.VectorSubcoreMesh(core_axis_name="c", subcore_axis_name="s")
  out_type = [jax.ShapeDtypeStruct((NC, NP, wr), jnp.float32) for _ in range(C)]
  scratch = [
      pltpu.VMEM((idxc, 128), jnp.int32),
      pltpu.VMEM((idxc, 128), jnp.int32),
      pltpu.VMEM((128, wr), jnp.float32),
      pltpu.VMEM((128, wr), jnp.float32),
      pltpu.VMEM((128, wr), jnp.float32),
      pltpu.VMEM_SHARED((NP, wr), jnp.float32),
      pltpu.VMEM_SHARED((NP, wr), jnp.float32),
      pltpu.SemaphoreType.DMA,
      pltpu.SemaphoreType.DMA,
  ]

  @functools.partial(pl.kernel, mesh=mesh, out_type=out_type,
                     scratch_types=scratch,
                     compiler_params=pltpu.CompilerParams(
                         use_tc_tiling_on_sc=False))
  def k(*refs):
    gs = refs[:C]
    src_hbm, dst_hbm, z_hbm = refs[C], refs[C + 1], refs[C + 2]
    outs = refs[C + 3:C + 3 + C]
    src_v, dst_v, buf0, buf1, zbuf, tab, acc, sem0, sem1 = refs[C + 3 + C:]

    cid = lax.axis_index("c")
    sid = lax.axis_index("s")
    wid = sid * NC + cid
    row0 = wid * rpt
    my_lo = sid * SLICE

    pltpu.sync_copy(z_hbm, zbuf)

    for c in range(C):
      pltpu.sync_copy(gs[c].at[pl.ds(my_lo, SLICE)], tab.at[pl.ds(my_lo, SLICE)])
      for z in range(SLICE // 128):
        pltpu.sync_copy(zbuf, acc.at[pl.ds(my_lo + z * 128, 128)])
      plsc.subcore_barrier()

      for st in range(rpt // idxc):
        pltpu.sync_copy(src_hbm.at[pl.ds(row0 + st * idxc, idxc)], src_v)
        pltpu.sync_copy(dst_hbm.at[pl.ds(row0 + st * idxc, idxc)], dst_v)
        pltpu.async_copy(tab.at[src_v.at[0]], buf0, sem0)

        def body(it, _):
          j = it * 2
          pltpu.make_async_copy(tab.at[src_v.at[j]], buf0, sem0).wait()
          pltpu.async_copy(tab.at[src_v.at[j + 1]], buf1, sem1)
          pltpu.sync_copy(buf0, acc.at[dst_v.at[j]], add=True)
          pltpu.make_async_copy(tab.at[src_v.at[j + 1]], buf1, sem1).wait()

          @pl.when(j + 2 < idxc)
          def _():
            pltpu.async_copy(tab.at[src_v.at[j + 2]], buf0, sem0)

          pltpu.sync_copy(buf1, acc.at[dst_v.at[j + 1]], add=True)
          return 0

        lax.fori_loop(0, idxc // 2, body, 0)
      plsc.subcore_barrier()

      pltpu.sync_copy(acc.at[pl.ds(my_lo, SLICE)],
                      outs[c].at[cid, pl.ds(my_lo, SLICE)])
      plsc.subcore_barrier()

  res = k(*g_list, src2d, dst2d, zrow)
  return list(res) if isinstance(res, (tuple, list)) else [res]


def _sc_degree(dst2d, ones_row, zrow):
  epr = dst2d.shape[0]
  rpt = epr // NW
  wr = ones_row.shape[1]
  mesh = plsc.VectorSubcoreMesh(core_axis_name="c", subcore_axis_name="s")
  scratch = [
      pltpu.VMEM((rpt, 128), jnp.int32),
      pltpu.VMEM((128, wr), jnp.float32),
      pltpu.VMEM((128, wr), jnp.float32),
      pltpu.VMEM_SHARED((NP, wr), jnp.float32),
  ]

  @functools.partial(
      pl.kernel, mesh=mesh,
      out_type=jax.ShapeDtypeStruct((NC, NP, wr), jnp.float32),
      scratch_types=scratch,
      compiler_params=pltpu.CompilerParams(use_tc_tiling_on_sc=False))
  def k(dst_hbm, ones_hbm, z_hbm, out_hbm, dst_v, obuf, zbuf, acc):
    cid = lax.axis_index("c")
    sid = lax.axis_index("s")
    wid = sid * NC + cid
    row0 = wid * rpt
    my_lo = sid * SLICE

    pltpu.sync_copy(dst_hbm.at[pl.ds(row0, rpt)], dst_v)
    pltpu.sync_copy(ones_hbm, obuf)
    pltpu.sync_copy(z_hbm, zbuf)
    for z in range(SLICE // 128):
      pltpu.sync_copy(zbuf, acc.at[pl.ds(my_lo + z * 128, 128)])
    plsc.subcore_barrier()

    def body(j, _):
      pltpu.sync_copy(obuf, acc.at[dst_v.at[j]], add=True)
      return 0

    lax.fori_loop(0, rpt, body, 0)
    plsc.subcore_barrier()
    pltpu.sync_copy(acc.at[pl.ds(my_lo, SLICE)],
                    out_hbm.at[cid, pl.ds(my_lo, SLICE)])

  return k(dst2d, ones_row, zrow)


def _tc_prep(deg_p, x_pad):
  grid = (NP // TN,)

  def body(deg_ref, x_ref, a_ref, g_ref):
    i = pl.program_id(0)
    rows = i * TN + lax.broadcasted_iota(jnp.int32, (TN, 1), 0)
    d = deg_ref[0][:, :1] + deg_ref[1][:, :1] + 1.0
    a = jnp.where(rows < N, lax.rsqrt(d), 0.0)
    a_ref[...] = a
    col0 = lax.broadcasted_iota(jnp.int32, (TN, W1R), 1) == 0
    g_ref[...] = jnp.where(col0, a * x_ref[...], 0.0)

  return pl.pallas_call(
      body, grid=grid,
      in_specs=[pl.BlockSpec((NC, TN, W1R), lambda i: (0, i, 0)),
                pl.BlockSpec((TN, 1), lambda i: (i, 0))],
      out_specs=[pl.BlockSpec((TN, 1), lambda i: (i, 0)),
                 pl.BlockSpec((TN, W1R), lambda i: (i, 0))],
      out_shape=[jax.ShapeDtypeStruct((NP, 1), jnp.float32),
                 jax.ShapeDtypeStruct((NP, W1R), jnp.float32)],
  )(deg_p, x_pad)


def _tc_layer1(P, g1, a, W1, b1):
  grid = (NP // TN,)

  def body(p_ref, g_ref, a_ref, w_ref, b_ref, *outs):
    av = a_ref[...]
    agg = av * (p_ref[0][:, :1] + p_ref[1][:, :1] + g_ref[:, :1])
    h = jnp.maximum(agg * w_ref[...] + b_ref[...], 0.0)
    gh = av * h
    for c in range(len(outs)):
      outs[c][...] = gh[:, c * CW:(c + 1) * CW]

  n_out = 32 // CW
  return pl.pallas_call(
      body, grid=grid,
      in_specs=[pl.BlockSpec((NC, TN, W1R), lambda i: (0, i, 0)),
                pl.BlockSpec((TN, W1R), lambda i: (i, 0)),
                pl.BlockSpec((TN, 1), lambda i: (i, 0)),
                pl.BlockSpec((1, 32), lambda i: (0, 0)),
                pl.BlockSpec((1, 32), lambda i: (0, 0))],
      out_specs=[pl.BlockSpec((TN, CW), lambda i: (i, 0))] * n_out,
      out_shape=[jax.ShapeDtypeStruct((NP, CW), jnp.float32)] * n_out,
  )(P, g1, a, W1, b1)


def _tc_layer(P_list, g_list, a, W, b, c_out, g_out):
  C = len(g_list)
  f_out = W.shape[1]
  grid = (NP // TN,)

  def body(*refs):
    p_refs = refs[:C]
    g_refs = refs[C:2 * C]
    a_ref, w_ref, b_ref = refs[2 * C], refs[2 * C + 1], refs[2 * C + 2]
    outs = refs[2 * C + 3:]
    av = a_ref[...]
    acc = None
    for c in range(C):
      pc = p_refs[c]
      aggc = av * (pc[0] + pc[1] + g_refs[c][...])
      part = jnp.dot(aggc, w_ref[c * CW:(c + 1) * CW, :],
                     preferred_element_type=jnp.float32)
      acc = part if acc is None else acc + part
    h = jnp.maximum(acc + b_ref[...], 0.0)
    if g_out:
      gh = av * h
      for c2 in range(len(outs)):
        outs[c2][...] = gh[:, c2 * CW:(c2 + 1) * CW]
    else:
      outs[0][...] = h

  in_specs = ([pl.BlockSpec((NC, TN, CW), lambda i: (0, i, 0))] * C +
              [pl.BlockSpec((TN, CW), lambda i: (i, 0))] * C +
              [pl.BlockSpec((TN, 1), lambda i: (i, 0)),
               pl.BlockSpec(W.shape, lambda i: (0, 0)),
               pl.BlockSpec((1, f_out), lambda i: (0, 0))])
  if g_out:
    out_specs = [pl.BlockSpec((TN, CW), lambda i: (i, 0))] * c_out
    out_shape = [jax.ShapeDtypeStruct((NP, CW), jnp.float32)] * c_out
  else:
    out_specs = [pl.BlockSpec((TN, f_out), lambda i: (i, 0))]
    out_shape = [jax.ShapeDtypeStruct((NP, f_out), jnp.float32)]
  res = pl.pallas_call(body, grid=grid, in_specs=in_specs,
                       out_specs=out_specs, out_shape=out_shape)(
                           *P_list, *g_list, a, W, b)
  return res


def _tc_pool_mlp(h5, batch3d, fcW1, fcb1, fcW2, fcb2):
  grid = (NP // TN,)
  nb = grid[0]

  def body(h_ref, b_ref, w1_ref, c1_ref, w2_ref, c2_ref, o_ref, s_ref, n_ref):
    i = pl.program_id(0)

    @pl.when(i == 0)
    def _():
      s_ref[...] = jnp.zeros_like(s_ref)
      n_ref[...] = jnp.zeros_like(n_ref)

    ids = b_ref[0]
    gids = lax.broadcasted_iota(jnp.int32, (G, TN), 0)
    mask = (gids == ids).astype(jnp.float32)
    s_ref[...] += jnp.dot(mask, h_ref[...], preferred_element_type=jnp.float32)
    n_ref[...] += jnp.sum(mask, axis=1, keepdims=True)

    @pl.when(i == nb - 1)
    def _():
      mean = s_ref[...] / jnp.clip(n_ref[...], 1.0, None)
      r = jnp.maximum(jnp.dot(mean, w1_ref[...],
                              preferred_element_type=jnp.float32) +
                      c1_ref[...], 0.0)
      o_ref[...] = jnp.dot(r, w2_ref[...],
                           preferred_element_type=jnp.float32) + c2_ref[...]

  return pl.pallas_call(
      body, grid=grid,
      in_specs=[pl.BlockSpec((TN, 256), lambda i: (i, 0)),
                pl.BlockSpec((1, 1, TN), lambda i: (i, 0, 0)),
                pl.BlockSpec((256, 128), lambda i: (0, 0)),
                pl.BlockSpec((1, 128), lambda i: (0, 0)),
                pl.BlockSpec((128, 10), lambda i: (0, 0)),
                pl.BlockSpec((1, 10), lambda i: (0, 0))],
      out_specs=pl.BlockSpec((G, 10), lambda i: (0, 0)),
      out_shape=jax.ShapeDtypeStruct((G, 10), jnp.float32),
      scratch_shapes=[pltpu.VMEM((G, 256), jnp.float32),
                      pltpu.VMEM((G, 1), jnp.float32)],
  )(h5, batch3d, fcW1, fcb1, fcW2, fcb2)


def kernel(x, edge_index, batch, W1, b1, W2, b2, W3, b3, W4, b4,
           fcW1, fcb1, fcW2, fcb2):
  E = edge_index.shape[1]
  rpt = (E + NW * 128 - 1) // (NW * 128)
  rpt = ((rpt + IDXC - 1) // IDXC) * IDXC
  epr = rpt * NW
  ep = epr * 128
  pad = ep - E
  padv = jnp.full((pad,), N, dtype=jnp.int32)
  src2d = jnp.concatenate([edge_index[0], padv]).reshape(epr, 128)
  dst2d = jnp.concatenate([edge_index[1], padv]).reshape(epr, 128)

  x_pad = jnp.pad(x, ((0, NP - N), (0, 0)))
  batch3d = jnp.pad(batch, (0, NP - N), constant_values=G).reshape(
      NP // TN, 1, TN)

  z1 = jnp.zeros((128, W1R), jnp.float32)
  o1 = jnp.ones((128, W1R), jnp.float32)
  zc = jnp.zeros((128, CW), jnp.float32)

  deg_p = _sc_degree(dst2d, o1, z1)
  a, g1 = _tc_prep(deg_p, x_pad)

  (P1,) = _sc_agg([g1], src2d, dst2d, z1, W1R)
  g2 = list(_tc_layer1(P1, g1, a, W1.astype(jnp.float32),
                       b1.reshape(1, 32).astype(jnp.float32)))

  P2 = _sc_agg(g2, src2d, dst2d, zc, CW)
  g3 = list(_tc_layer(P2, g2, a, W2, b2.reshape(1, 64), 64 // CW, True))

  P3 = _sc_agg(g3, src2d, dst2d, zc, CW)
  g4 = list(_tc_layer(P3, g3, a, W3, b3.reshape(1, 128), 128 // CW, True))

  P4 = _sc_agg(g4, src2d, dst2d, zc, CW)
  (h5,) = _tc_layer(P4, g4, a, W4, b4.reshape(1, 256), 0, False)

  return _tc_pool_mlp(h5, batch3d, fcW1, fcb1.reshape(1, 128),
                      fcW2, fcb2.reshape(1, 10))


# speedup vs baseline: 14.7129x; 1.0376x over previous
import functools

import jax
import jax.numpy as jnp
from jax import lax
from jax.experimental import pallas as pl
from jax.experimental.pallas import tpu as pltpu
from jax.experimental.pallas import tpu_sc as plsc

N = 50000
G = 64
NC = 2
NS = 16
NW = NC * NS
NP = 51200
SLICE = NP // NS
CW = 16
IDXC = 40
TN = 512
W1R = 8


def _sc_agg(g, src2d, dst2d, zrow):
  C, _, wr = g.shape
  epr = src2d.shape[0]
  rpt = epr // NW
  idxc = IDXC
  mesh = plsc.VectorSubcoreMesh(core_axis_name="c", subcore_axis_name="s")
  out_type = jax.ShapeDtypeStruct((C, NC, NP, wr), jnp.float32)
  scratch = [
      pltpu.VMEM((idxc, 128), jnp.int32),
      pltpu.VMEM((idxc, 128), jnp.int32),
      pltpu.VMEM((128, wr), jnp.float32),
      pltpu.VMEM((128, wr), jnp.float32),
      pltpu.VMEM((128, wr), jnp.float32),
      pltpu.VMEM_SHARED((NP, wr), jnp.float32),
      pltpu.VMEM_SHARED((NP, wr), jnp.float32),
      pltpu.SemaphoreType.DMA,
      pltpu.SemaphoreType.DMA,
  ]

  @functools.partial(pl.kernel, mesh=mesh, out_type=out_type,
                     scratch_types=scratch,
                     compiler_params=pltpu.CompilerParams(
                         use_tc_tiling_on_sc=False))
  def k(g_hbm, src_hbm, dst_hbm, z_hbm, out_hbm,
        src_v, dst_v, buf0, buf1, zbuf, tab, acc, sem0, sem1):
    cid = lax.axis_index("c")
    sid = lax.axis_index("s")
    wid = sid * NC + cid
    row0 = wid * rpt
    my_lo = sid * SLICE

    pltpu.sync_copy(z_hbm, zbuf)

    for c in range(C):
      pltpu.sync_copy(g_hbm.at[c, pl.ds(my_lo, SLICE)],
                      tab.at[pl.ds(my_lo, SLICE)])
      for z in range(SLICE // 128):
        pltpu.sync_copy(zbuf, acc.at[pl.ds(my_lo + z * 128, 128)])
      plsc.subcore_barrier()

      for st in range(rpt // idxc):
        pltpu.sync_copy(src_hbm.at[pl.ds(row0 + st * idxc, idxc)], src_v)
        pltpu.sync_copy(dst_hbm.at[pl.ds(row0 + st * idxc, idxc)], dst_v)
        pltpu.async_copy(tab.at[src_v.at[0]], buf0, sem0)

        def body(it, _):
          j = it * 2
          pltpu.make_async_copy(tab.at[src_v.at[j]], buf0, sem0).wait()
          pltpu.async_copy(tab.at[src_v.at[j + 1]], buf1, sem1)
          pltpu.sync_copy(buf0, acc.at[dst_v.at[j]], add=True)
          pltpu.make_async_copy(tab.at[src_v.at[j + 1]], buf1, sem1).wait()

          @pl.when(j + 2 < idxc)
          def _():
            pltpu.async_copy(tab.at[src_v.at[j + 2]], buf0, sem0)

          pltpu.sync_copy(buf1, acc.at[dst_v.at[j + 1]], add=True)
          return 0

        lax.fori_loop(0, idxc // 2, body, 0)
      plsc.subcore_barrier()

      pltpu.sync_copy(acc.at[pl.ds(my_lo, SLICE)],
                      out_hbm.at[c, cid, pl.ds(my_lo, SLICE)])
      plsc.subcore_barrier()

  return k(g, src2d, dst2d, zrow)


def _sc_degree(dst2d, ones_row, zrow):
  epr = dst2d.shape[0]
  rpt = epr // NW
  wr = ones_row.shape[1]
  mesh = plsc.VectorSubcoreMesh(core_axis_name="c", subcore_axis_name="s")
  scratch = [
      pltpu.VMEM((rpt, 128), jnp.int32),
      pltpu.VMEM((128, wr), jnp.float32),
      pltpu.VMEM((128, wr), jnp.float32),
      pltpu.VMEM_SHARED((NP, wr), jnp.float32),
  ]

  @functools.partial(
      pl.kernel, mesh=mesh,
      out_type=jax.ShapeDtypeStruct((NC, NP, wr), jnp.float32),
      scratch_types=scratch,
      compiler_params=pltpu.CompilerParams(use_tc_tiling_on_sc=False))
  def k(dst_hbm, ones_hbm, z_hbm, out_hbm, dst_v, obuf, zbuf, acc):
    cid = lax.axis_index("c")
    sid = lax.axis_index("s")
    wid = sid * NC + cid
    row0 = wid * rpt
    my_lo = sid * SLICE

    pltpu.sync_copy(dst_hbm.at[pl.ds(row0, rpt)], dst_v)
    pltpu.sync_copy(ones_hbm, obuf)
    pltpu.sync_copy(z_hbm, zbuf)
    for z in range(SLICE // 128):
      pltpu.sync_copy(zbuf, acc.at[pl.ds(my_lo + z * 128, 128)])
    plsc.subcore_barrier()

    def body(j, _):
      pltpu.sync_copy(obuf, acc.at[dst_v.at[j]], add=True)
      return 0

    lax.fori_loop(0, rpt, body, 0)
    plsc.subcore_barrier()
    pltpu.sync_copy(acc.at[pl.ds(my_lo, SLICE)],
                    out_hbm.at[cid, pl.ds(my_lo, SLICE)])

  return k(dst2d, ones_row, zrow)


def _tc_prep(deg_p, x_pad):
  grid = (NP // TN,)

  def body(deg_ref, x_ref, a_ref, g_ref):
    i = pl.program_id(0)
    rows = i * TN + lax.broadcasted_iota(jnp.int32, (TN, 1), 0)
    d = deg_ref[0][:, :1] + deg_ref[1][:, :1] + 1.0
    a = jnp.where(rows < N, lax.rsqrt(d), 0.0)
    a_ref[...] = a
    col0 = lax.broadcasted_iota(jnp.int32, (TN, W1R), 1) == 0
    g_ref[0] = jnp.where(col0, a * x_ref[...], 0.0)

  return pl.pallas_call(
      body, grid=grid,
      in_specs=[pl.BlockSpec((NC, TN, W1R), lambda i: (0, i, 0)),
                pl.BlockSpec((TN, 1), lambda i: (i, 0))],
      out_specs=[pl.BlockSpec((TN, 1), lambda i: (i, 0)),
                 pl.BlockSpec((1, TN, W1R), lambda i: (0, i, 0))],
      out_shape=[jax.ShapeDtypeStruct((NP, 1), jnp.float32),
                 jax.ShapeDtypeStruct((1, NP, W1R), jnp.float32)],
  )(deg_p, x_pad)


def _tc_layer1(P, g1, a, W1, b1):
  grid = (NP // TN,)
  n_out = 32 // CW

  def body(p_ref, g_ref, a_ref, w_ref, b_ref, o_ref):
    av = a_ref[...]
    agg = av * (p_ref[0, 0][:, :1] + p_ref[0, 1][:, :1] + g_ref[0][:, :1])
    h = jnp.maximum(agg * w_ref[...] + b_ref[...], 0.0)
    gh = av * h
    for c in range(n_out):
      o_ref[c] = gh[:, c * CW:(c + 1) * CW]

  return pl.pallas_call(
      body, grid=grid,
      in_specs=[pl.BlockSpec((1, NC, TN, W1R), lambda i: (0, 0, i, 0)),
                pl.BlockSpec((1, TN, W1R), lambda i: (0, i, 0)),
                pl.BlockSpec((TN, 1), lambda i: (i, 0)),
                pl.BlockSpec((1, 32), lambda i: (0, 0)),
                pl.BlockSpec((1, 32), lambda i: (0, 0))],
      out_specs=pl.BlockSpec((n_out, TN, CW), lambda i: (0, i, 0)),
      out_shape=jax.ShapeDtypeStruct((n_out, NP, CW), jnp.float32),
  )(P, g1, a, W1, b1)


def _tc_layer(P, g, a, W, b, c_out):
  C = g.shape[0]
  f_out = W.shape[1]
  grid = (NP // TN,)

  def body(p_ref, g_ref, a_ref, w_ref, b_ref, o_ref):
    av = a_ref[...]
    acc = None
    for c in range(C):
      aggc = av * (p_ref[c, 0] + p_ref[c, 1] + g_ref[c])
      part = jnp.dot(aggc, w_ref[c * CW:(c + 1) * CW, :],
                     preferred_element_type=jnp.float32)
      acc = part if acc is None else acc + part
    h = jnp.maximum(acc + b_ref[...], 0.0)
    gh = av * h
    for c2 in range(c_out):
      o_ref[c2] = gh[:, c2 * CW:(c2 + 1) * CW]

  return pl.pallas_call(
      body, grid=grid,
      in_specs=[pl.BlockSpec((C, NC, TN, CW), lambda i: (0, 0, i, 0)),
                pl.BlockSpec((C, TN, CW), lambda i: (0, i, 0)),
                pl.BlockSpec((TN, 1), lambda i: (i, 0)),
                pl.BlockSpec(W.shape, lambda i: (0, 0)),
                pl.BlockSpec((1, f_out), lambda i: (0, 0))],
      out_specs=pl.BlockSpec((c_out, TN, CW), lambda i: (0, i, 0)),
      out_shape=jax.ShapeDtypeStruct((c_out, NP, CW), jnp.float32),
  )(P, g, a, W, b)


def _tc_layer_pool(P, g, a, W, b, batch3d, fcW1, fcb1, fcW2, fcb2):
  C = g.shape[0]
  f_out = W.shape[1]
  grid = (NP // TN,)
  nb = grid[0]

  def body(p_ref, g_ref, a_ref, w_ref, b_ref, bt_ref,
           w1_ref, c1_ref, w2_ref, c2_ref, o_ref, s_ref, n_ref):
    i = pl.program_id(0)

    @pl.when(i == 0)
    def _():
      s_ref[...] = jnp.zeros_like(s_ref)
      n_ref[...] = jnp.zeros_like(n_ref)

    av = a_ref[...]
    acc = None
    for c in range(C):
      aggc = av * (p_ref[c, 0] + p_ref[c, 1] + g_ref[c])
      part = jnp.dot(aggc, w_ref[c * CW:(c + 1) * CW, :],
                     preferred_element_type=jnp.float32)
      acc = part if acc is None else acc + part
    h = jnp.maximum(acc + b_ref[...], 0.0)

    ids = bt_ref[0]
    gids = lax.broadcasted_iota(jnp.int32, (G, TN), 0)
    mask = (gids == ids).astype(jnp.float32)
    s_ref[...] += jnp.dot(mask, h, preferred_element_type=jnp.float32)
    n_ref[...] += jnp.sum(mask, axis=1, keepdims=True)

    @pl.when(i == nb - 1)
    def _():
      mean = s_ref[...] / jnp.clip(n_ref[...], 1.0, None)
      r = jnp.maximum(jnp.dot(mean, w1_ref[...],
                              preferred_element_type=jnp.float32) +
                      c1_ref[...], 0.0)
      o_ref[...] = jnp.dot(r, w2_ref[...],
                           preferred_element_type=jnp.float32) + c2_ref[...]

  return pl.pallas_call(
      body, grid=grid,
      in_specs=[pl.BlockSpec((C, NC, TN, CW), lambda i: (0, 0, i, 0)),
                pl.BlockSpec((C, TN, CW), lambda i: (0, i, 0)),
                pl.BlockSpec((TN, 1), lambda i: (i, 0)),
                pl.BlockSpec(W.shape, lambda i: (0, 0)),
                pl.BlockSpec((1, f_out), lambda i: (0, 0)),
                pl.BlockSpec((1, 1, TN), lambda i: (i, 0, 0)),
                pl.BlockSpec((256, 128), lambda i: (0, 0)),
                pl.BlockSpec((1, 128), lambda i: (0, 0)),
                pl.BlockSpec((128, 10), lambda i: (0, 0)),
                pl.BlockSpec((1, 10), lambda i: (0, 0))],
      out_specs=pl.BlockSpec((G, 10), lambda i: (0, 0)),
      out_shape=jax.ShapeDtypeStruct((G, 10), jnp.float32),
      scratch_shapes=[pltpu.VMEM((G, 256), jnp.float32),
                      pltpu.VMEM((G, 1), jnp.float32)],
  )(P, g, a, W, b, batch3d, fcW1, fcb1, fcW2, fcb2)


def kernel(x, edge_index, batch, W1, b1, W2, b2, W3, b3, W4, b4,
           fcW1, fcb1, fcW2, fcb2):
  E = edge_index.shape[1]
  rpt = (E + NW * 128 - 1) // (NW * 128)
  rpt = ((rpt + IDXC - 1) // IDXC) * IDXC
  epr = rpt * NW
  ep = epr * 128
  pad = ep - E
  padv = jnp.full((pad,), N, dtype=jnp.int32)
  src2d = jnp.concatenate([edge_index[0], padv]).reshape(epr, 128)
  dst2d = jnp.concatenate([edge_index[1], padv]).reshape(epr, 128)

  x_pad = jnp.pad(x, ((0, NP - N), (0, 0)))
  batch3d = jnp.pad(batch, (0, NP - N), constant_values=G).reshape(
      NP // TN, 1, TN)

  z1 = jnp.zeros((128, W1R), jnp.float32)
  o1 = jnp.ones((128, W1R), jnp.float32)
  zc = jnp.zeros((128, CW), jnp.float32)

  deg_p = _sc_degree(dst2d, o1, z1)
  a, g1 = _tc_prep(deg_p, x_pad)

  P1 = _sc_agg(g1, src2d, dst2d, z1)
  g2 = _tc_layer1(P1, g1, a, W1.astype(jnp.float32),
                  b1.reshape(1, 32).astype(jnp.float32))

  P2 = _sc_agg(g2, src2d, dst2d, zc)
  g3 = _tc_layer(P2, g2, a, W2, b2.reshape(1, 64), 64 // CW)

  P3 = _sc_agg(g3, src2d, dst2d, zc)
  g4 = _tc_layer(P3, g3, a, W3, b3.reshape(1, 128), 128 // CW)

  P4 = _sc_agg(g4, src2d, dst2d, zc)
  return _tc_layer_pool(P4, g4, a, W4, b4.reshape(1, 256), batch3d,
                        fcW1, fcb1.reshape(1, 128), fcW2, fcb2.reshape(1, 10))


# speedup vs baseline: 14.9639x; 1.0171x over previous
import functools

import jax
import jax.numpy as jnp
from jax import lax
from jax.experimental import pallas as pl
from jax.experimental.pallas import tpu as pltpu
from jax.experimental.pallas import tpu_sc as plsc

N = 50000
G = 64
NC = 2
NS = 16
NW = NC * NS
NP = 51200
SLICE = NP // NS
CW = 16
IDXC = 50
TN = 512
W1R = 8


def _sc_agg(g, src2d, dst2d, zrow):
  C, _, wr = g.shape
  epr = src2d.shape[0]
  rpt = epr // NW
  idxc = IDXC
  mesh = plsc.VectorSubcoreMesh(core_axis_name="c", subcore_axis_name="s")
  out_type = jax.ShapeDtypeStruct((C, NC, NP, wr), jnp.float32)
  scratch = [
      pltpu.VMEM((idxc, 128), jnp.int32),
      pltpu.VMEM((idxc, 128), jnp.int32),
      pltpu.VMEM((128, wr), jnp.float32),
      pltpu.VMEM((128, wr), jnp.float32),
      pltpu.VMEM((128, wr), jnp.float32),
      pltpu.VMEM_SHARED((NP, wr), jnp.float32),
      pltpu.VMEM_SHARED((NP, wr), jnp.float32),
      pltpu.SemaphoreType.DMA,
      pltpu.SemaphoreType.DMA,
  ]

  @functools.partial(pl.kernel, mesh=mesh, out_type=out_type,
                     scratch_types=scratch,
                     compiler_params=pltpu.CompilerParams(
                         use_tc_tiling_on_sc=False))
  def k(g_hbm, src_hbm, dst_hbm, z_hbm, out_hbm,
        src_v, dst_v, buf0, buf1, zbuf, tab, acc, sem0, sem1):
    cid = lax.axis_index("c")
    sid = lax.axis_index("s")
    wid = sid * NC + cid
    row0 = wid * rpt
    my_lo = sid * SLICE

    pltpu.sync_copy(z_hbm, zbuf)

    for c in range(C):
      pltpu.sync_copy(g_hbm.at[c, pl.ds(my_lo, SLICE)],
                      tab.at[pl.ds(my_lo, SLICE)])
      for z in range(SLICE // 128):
        pltpu.sync_copy(zbuf, acc.at[pl.ds(my_lo + z * 128, 128)])
      plsc.subcore_barrier()

      for st in range(rpt // idxc):
        pltpu.sync_copy(src_hbm.at[pl.ds(row0 + st * idxc, idxc)], src_v)
        pltpu.sync_copy(dst_hbm.at[pl.ds(row0 + st * idxc, idxc)], dst_v)
        pltpu.async_copy(tab.at[src_v.at[0]], buf0, sem0)

        def body(it, _):
          j = it * 2
          pltpu.make_async_copy(tab.at[src_v.at[j]], buf0, sem0).wait()
          pltpu.async_copy(tab.at[src_v.at[j + 1]], buf1, sem1)
          pltpu.sync_copy(buf0, acc.at[dst_v.at[j]], add=True)
          pltpu.make_async_copy(tab.at[src_v.at[j + 1]], buf1, sem1).wait()

          @pl.when(j + 2 < idxc)
          def _():
            pltpu.async_copy(tab.at[src_v.at[j + 2]], buf0, sem0)

          pltpu.sync_copy(buf1, acc.at[dst_v.at[j + 1]], add=True)
          return 0

        lax.fori_loop(0, idxc // 2, body, 0)
      plsc.subcore_barrier()

      pltpu.sync_copy(acc.at[pl.ds(my_lo, SLICE)],
                      out_hbm.at[c, cid, pl.ds(my_lo, SLICE)])

  return k(g, src2d, dst2d, zrow)


def _sc_degree(dst2d, ones_row, zrow):
  epr = dst2d.shape[0]
  rpt = epr // NW
  wr = ones_row.shape[1]
  mesh = plsc.VectorSubcoreMesh(core_axis_name="c", subcore_axis_name="s")
  scratch = [
      pltpu.VMEM((rpt, 128), jnp.int32),
      pltpu.VMEM((128, wr), jnp.float32),
      pltpu.VMEM((128, wr), jnp.float32),
      pltpu.VMEM_SHARED((NP, wr), jnp.float32),
  ]

  @functools.partial(
      pl.kernel, mesh=mesh,
      out_type=jax.ShapeDtypeStruct((NC, NP, wr), jnp.float32),
      scratch_types=scratch,
      compiler_params=pltpu.CompilerParams(use_tc_tiling_on_sc=False))
  def k(dst_hbm, ones_hbm, z_hbm, out_hbm, dst_v, obuf, zbuf, acc):
    cid = lax.axis_index("c")
    sid = lax.axis_index("s")
    wid = sid * NC + cid
    row0 = wid * rpt
    my_lo = sid * SLICE

    pltpu.sync_copy(dst_hbm.at[pl.ds(row0, rpt)], dst_v)
    pltpu.sync_copy(ones_hbm, obuf)
    pltpu.sync_copy(z_hbm, zbuf)
    for z in range(SLICE // 128):
      pltpu.sync_copy(zbuf, acc.at[pl.ds(my_lo + z * 128, 128)])
    plsc.subcore_barrier()

    def body(j, _):
      pltpu.sync_copy(obuf, acc.at[dst_v.at[j]], add=True)
      return 0

    lax.fori_loop(0, rpt, body, 0)
    plsc.subcore_barrier()
    pltpu.sync_copy(acc.at[pl.ds(my_lo, SLICE)],
                    out_hbm.at[cid, pl.ds(my_lo, SLICE)])

  return k(dst2d, ones_row, zrow)


def _tc_prep(deg_p, x_pad):
  grid = (NP // TN,)

  def body(deg_ref, x_ref, a_ref, g_ref):
    i = pl.program_id(0)
    rows = i * TN + lax.broadcasted_iota(jnp.int32, (TN, 1), 0)
    d = deg_ref[0][:, :1] + deg_ref[1][:, :1] + 1.0
    a = jnp.where(rows < N, lax.rsqrt(d), 0.0)
    a_ref[...] = a
    col0 = lax.broadcasted_iota(jnp.int32, (TN, W1R), 1) == 0
    g_ref[0] = jnp.where(col0, a * x_ref[...], 0.0)

  return pl.pallas_call(
      body, grid=grid,
      in_specs=[pl.BlockSpec((NC, TN, W1R), lambda i: (0, i, 0)),
                pl.BlockSpec((TN, 1), lambda i: (i, 0))],
      out_specs=[pl.BlockSpec((TN, 1), lambda i: (i, 0)),
                 pl.BlockSpec((1, TN, W1R), lambda i: (0, i, 0))],
      out_shape=[jax.ShapeDtypeStruct((NP, 1), jnp.float32),
                 jax.ShapeDtypeStruct((1, NP, W1R), jnp.float32)],
  )(deg_p, x_pad)


def _tc_layer1(P, g1, a, W1, b1):
  grid = (NP // TN,)
  n_out = 32 // CW

  def body(p_ref, g_ref, a_ref, w_ref, b_ref, o_ref):
    av = a_ref[...]
    agg = av * (p_ref[0, 0][:, :1] + p_ref[0, 1][:, :1] + g_ref[0][:, :1])
    h = jnp.maximum(agg * w_ref[...] + b_ref[...], 0.0)
    gh = av * h
    for c in range(n_out):
      o_ref[c] = gh[:, c * CW:(c + 1) * CW]

  return pl.pallas_call(
      body, grid=grid,
      in_specs=[pl.BlockSpec((1, NC, TN, W1R), lambda i: (0, 0, i, 0)),
                pl.BlockSpec((1, TN, W1R), lambda i: (0, i, 0)),
                pl.BlockSpec((TN, 1), lambda i: (i, 0)),
                pl.BlockSpec((1, 32), lambda i: (0, 0)),
                pl.BlockSpec((1, 32), lambda i: (0, 0))],
      out_specs=pl.BlockSpec((n_out, TN, CW), lambda i: (0, i, 0)),
      out_shape=jax.ShapeDtypeStruct((n_out, NP, CW), jnp.float32),
  )(P, g1, a, W1, b1)


def _tc_layer(P, g, a, W, b, c_out):
  C = g.shape[0]
  f_out = W.shape[1]
  grid = (NP // TN,)

  def body(p_ref, g_ref, a_ref, w_ref, b_ref, o_ref):
    av = a_ref[...]
    agg = jnp.concatenate(
        [av * (p_ref[c, 0] + p_ref[c, 1] + g_ref[c]) for c in range(C)],
        axis=1)
    h = jnp.maximum(jnp.dot(agg, w_ref[...],
                            preferred_element_type=jnp.float32) +
                    b_ref[...], 0.0)
    gh = av * h
    for c2 in range(c_out):
      o_ref[c2] = gh[:, c2 * CW:(c2 + 1) * CW]

  return pl.pallas_call(
      body, grid=grid,
      in_specs=[pl.BlockSpec((C, NC, TN, CW), lambda i: (0, 0, i, 0)),
                pl.BlockSpec((C, TN, CW), lambda i: (0, i, 0)),
                pl.BlockSpec((TN, 1), lambda i: (i, 0)),
                pl.BlockSpec(W.shape, lambda i: (0, 0)),
                pl.BlockSpec((1, f_out), lambda i: (0, 0))],
      out_specs=pl.BlockSpec((c_out, TN, CW), lambda i: (0, i, 0)),
      out_shape=jax.ShapeDtypeStruct((c_out, NP, CW), jnp.float32),
  )(P, g, a, W, b)


def _tc_layer_pool(P, g, a, W, b, batch3d, fcW1, fcb1, fcW2, fcb2):
  C = g.shape[0]
  f_out = W.shape[1]
  grid = (NP // TN,)
  nb = grid[0]

  def body(p_ref, g_ref, a_ref, w_ref, b_ref, bt_ref,
           w1_ref, c1_ref, w2_ref, c2_ref, o_ref, s_ref, n_ref):
    i = pl.program_id(0)

    @pl.when(i == 0)
    def _():
      s_ref[...] = jnp.zeros_like(s_ref)
      n_ref[...] = jnp.zeros_like(n_ref)

    av = a_ref[...]
    agg = jnp.concatenate(
        [av * (p_ref[c, 0] + p_ref[c, 1] + g_ref[c]) for c in range(C)],
        axis=1)
    h = jnp.maximum(jnp.dot(agg, w_ref[...],
                            preferred_element_type=jnp.float32) +
                    b_ref[...], 0.0)

    ids = bt_ref[0]
    gids = lax.broadcasted_iota(jnp.int32, (G, TN), 0)
    mask = (gids == ids).astype(jnp.float32)
    s_ref[...] += jnp.dot(mask, h, preferred_element_type=jnp.float32)
    n_ref[...] += jnp.sum(mask, axis=1, keepdims=True)

    @pl.when(i == nb - 1)
    def _():
      mean = s_ref[...] / jnp.clip(n_ref[...], 1.0, None)
      r = jnp.maximum(jnp.dot(mean, w1_ref[...],
                              preferred_element_type=jnp.float32) +
                      c1_ref[...], 0.0)
      o_ref[...] = jnp.dot(r, w2_ref[...],
                           preferred_element_type=jnp.float32) + c2_ref[...]

  return pl.pallas_call(
      body, grid=grid,
      in_specs=[pl.BlockSpec((C, NC, TN, CW), lambda i: (0, 0, i, 0)),
                pl.BlockSpec((C, TN, CW), lambda i: (0, i, 0)),
                pl.BlockSpec((TN, 1), lambda i: (i, 0)),
                pl.BlockSpec(W.shape, lambda i: (0, 0)),
                pl.BlockSpec((1, f_out), lambda i: (0, 0)),
                pl.BlockSpec((1, 1, TN), lambda i: (i, 0, 0)),
                pl.BlockSpec((256, 128), lambda i: (0, 0)),
                pl.BlockSpec((1, 128), lambda i: (0, 0)),
                pl.BlockSpec((128, 10), lambda i: (0, 0)),
                pl.BlockSpec((1, 10), lambda i: (0, 0))],
      out_specs=pl.BlockSpec((G, 10), lambda i: (0, 0)),
      out_shape=jax.ShapeDtypeStruct((G, 10), jnp.float32),
      scratch_shapes=[pltpu.VMEM((G, 256), jnp.float32),
                      pltpu.VMEM((G, 1), jnp.float32)],
  )(P, g, a, W, b, batch3d, fcW1, fcb1, fcW2, fcb2)


def kernel(x, edge_index, batch, W1, b1, W2, b2, W3, b3, W4, b4,
           fcW1, fcb1, fcW2, fcb2):
  E = edge_index.shape[1]
  rpt = (E + NW * 128 - 1) // (NW * 128)
  rpt = ((rpt + IDXC - 1) // IDXC) * IDXC
  epr = rpt * NW
  ep = epr * 128
  pad = ep - E
  padv = jnp.full((pad,), N, dtype=jnp.int32)
  src2d = jnp.concatenate([edge_index[0], padv]).reshape(epr, 128)
  dst2d = jnp.concatenate([edge_index[1], padv]).reshape(epr, 128)

  x_pad = jnp.pad(x, ((0, NP - N), (0, 0)))
  batch3d = jnp.pad(batch, (0, NP - N), constant_values=G).reshape(
      NP // TN, 1, TN)

  z1 = jnp.zeros((128, W1R), jnp.float32)
  o1 = jnp.ones((128, W1R), jnp.float32)
  zc = jnp.zeros((128, CW), jnp.float32)

  deg_p = _sc_degree(dst2d, o1, z1)
  a, g1 = _tc_prep(deg_p, x_pad)

  P1 = _sc_agg(g1, src2d, dst2d, z1)
  g2 = _tc_layer1(P1, g1, a, W1.astype(jnp.float32),
                  b1.reshape(1, 32).astype(jnp.float32))

  P2 = _sc_agg(g2, src2d, dst2d, zc)
  g3 = _tc_layer(P2, g2, a, W2, b2.reshape(1, 64), 64 // CW)

  P3 = _sc_agg(g3, src2d, dst2d, zc)
  g4 = _tc_layer(P3, g3, a, W3, b3.reshape(1, 128), 128 // CW)

  P4 = _sc_agg(g4, src2d, dst2d, zc)
  return _tc_layer_pool(P4, g4, a, W4, b4.reshape(1, 256), batch3d,
                        fcW1, fcb1.reshape(1, 128), fcW2, fcb2.reshape(1, 10))


# speedup vs baseline: 15.6245x; 1.0441x over previous
import functools

import jax
import jax.numpy as jnp
from jax import lax
from jax.experimental import pallas as pl
from jax.experimental.pallas import tpu as pltpu
from jax.experimental.pallas import tpu_sc as plsc

N = 50000
G = 64
NC = 2
NS = 16
NW = NC * NS
NP = 51200
SLICE = NP // NS
CW = 16
IDXC = 50
TN = 1024
W1R = 8


def _sc_agg(g, src2d, dst2d, zrow):
  C, _, wr = g.shape
  epr = src2d.shape[0]
  rpt = epr // NW
  idxc = IDXC
  mesh = plsc.VectorSubcoreMesh(core_axis_name="c", subcore_axis_name="s")
  out_type = jax.ShapeDtypeStruct((C, NC, NP, wr), jnp.float32)
  scratch = [
      pltpu.VMEM((idxc, 128), jnp.int32),
      pltpu.VMEM((idxc, 128), jnp.int32),
      pltpu.VMEM((128, wr), jnp.float32),
      pltpu.VMEM((128, wr), jnp.float32),
      pltpu.VMEM((128, wr), jnp.float32),
      pltpu.VMEM_SHARED((NP, wr), jnp.float32),
      pltpu.VMEM_SHARED((NP, wr), jnp.float32),
      pltpu.SemaphoreType.DMA,
      pltpu.SemaphoreType.DMA,
  ]

  @functools.partial(pl.kernel, mesh=mesh, out_type=out_type,
                     scratch_types=scratch,
                     compiler_params=pltpu.CompilerParams(
                         use_tc_tiling_on_sc=False))
  def k(g_hbm, src_hbm, dst_hbm, z_hbm, out_hbm,
        src_v, dst_v, buf0, buf1, zbuf, tab, acc, sem0, sem1):
    cid = lax.axis_index("c")
    sid = lax.axis_index("s")
    wid = sid * NC + cid
    row0 = wid * rpt
    my_lo = sid * SLICE

    pltpu.sync_copy(z_hbm, zbuf)

    for c in range(C):
      pltpu.sync_copy(g_hbm.at[c, pl.ds(my_lo, SLICE)],
                      tab.at[pl.ds(my_lo, SLICE)])
      for z in range(SLICE // 128):
        pltpu.sync_copy(zbuf, acc.at[pl.ds(my_lo + z * 128, 128)])
      plsc.subcore_barrier()

      for st in range(rpt // idxc):
        pltpu.sync_copy(src_hbm.at[pl.ds(row0 + st * idxc, idxc)], src_v)
        pltpu.sync_copy(dst_hbm.at[pl.ds(row0 + st * idxc, idxc)], dst_v)
        pltpu.async_copy(tab.at[src_v.at[0]], buf0, sem0)

        def body(it, _):
          j = it * 2
          pltpu.make_async_copy(tab.at[src_v.at[j]], buf0, sem0).wait()
          pltpu.async_copy(tab.at[src_v.at[j + 1]], buf1, sem1)
          pltpu.sync_copy(buf0, acc.at[dst_v.at[j]], add=True)
          pltpu.make_async_copy(tab.at[src_v.at[j + 1]], buf1, sem1).wait()

          @pl.when(j + 2 < idxc)
          def _():
            pltpu.async_copy(tab.at[src_v.at[j + 2]], buf0, sem0)

          pltpu.sync_copy(buf1, acc.at[dst_v.at[j + 1]], add=True)
          return 0

        lax.fori_loop(0, idxc // 2, body, 0)
      plsc.subcore_barrier()

      pltpu.sync_copy(acc.at[pl.ds(my_lo, SLICE)],
                      out_hbm.at[c, cid, pl.ds(my_lo, SLICE)])

  return k(g, src2d, dst2d, zrow)


def _sc_degree(dst2d, ones_row, zrow):
  epr = dst2d.shape[0]
  rpt = epr // NW
  wr = ones_row.shape[1]
  mesh = plsc.VectorSubcoreMesh(core_axis_name="c", subcore_axis_name="s")
  scratch = [
      pltpu.VMEM((rpt, 128), jnp.int32),
      pltpu.VMEM((128, wr), jnp.float32),
      pltpu.VMEM((128, wr), jnp.float32),
      pltpu.VMEM_SHARED((NP, wr), jnp.float32),
  ]

  @functools.partial(
      pl.kernel, mesh=mesh,
      out_type=jax.ShapeDtypeStruct((NC, NP, wr), jnp.float32),
      scratch_types=scratch,
      compiler_params=pltpu.CompilerParams(use_tc_tiling_on_sc=False))
  def k(dst_hbm, ones_hbm, z_hbm, out_hbm, dst_v, obuf, zbuf, acc):
    cid = lax.axis_index("c")
    sid = lax.axis_index("s")
    wid = sid * NC + cid
    row0 = wid * rpt
    my_lo = sid * SLICE

    pltpu.sync_copy(dst_hbm.at[pl.ds(row0, rpt)], dst_v)
    pltpu.sync_copy(ones_hbm, obuf)
    pltpu.sync_copy(z_hbm, zbuf)
    for z in range(SLICE // 128):
      pltpu.sync_copy(zbuf, acc.at[pl.ds(my_lo + z * 128, 128)])
    plsc.subcore_barrier()

    def body(j, _):
      pltpu.sync_copy(obuf, acc.at[dst_v.at[j]], add=True)
      return 0

    lax.fori_loop(0, rpt, body, 0)
    plsc.subcore_barrier()
    pltpu.sync_copy(acc.at[pl.ds(my_lo, SLICE)],
                    out_hbm.at[cid, pl.ds(my_lo, SLICE)])

  return k(dst2d, ones_row, zrow)


def _tc_prep(deg_p, x_pad):
  grid = (NP // TN,)

  def body(deg_ref, x_ref, a_ref, g_ref):
    i = pl.program_id(0)
    rows = i * TN + lax.broadcasted_iota(jnp.int32, (TN, 1), 0)
    d = deg_ref[0][:, :1] + deg_ref[1][:, :1] + 1.0
    a = jnp.where(rows < N, lax.rsqrt(d), 0.0)
    a_ref[...] = a
    col0 = lax.broadcasted_iota(jnp.int32, (TN, W1R), 1) == 0
    g_ref[0] = jnp.where(col0, a * x_ref[...], 0.0)

  return pl.pallas_call(
      body, grid=grid,
      in_specs=[pl.BlockSpec((NC, TN, W1R), lambda i: (0, i, 0)),
                pl.BlockSpec((TN, 1), lambda i: (i, 0))],
      out_specs=[pl.BlockSpec((TN, 1), lambda i: (i, 0)),
                 pl.BlockSpec((1, TN, W1R), lambda i: (0, i, 0))],
      out_shape=[jax.ShapeDtypeStruct((NP, 1), jnp.float32),
                 jax.ShapeDtypeStruct((1, NP, W1R), jnp.float32)],
  )(deg_p, x_pad)


def _tc_layer1(P, g1, a, W1, b1):
  grid = (NP // TN,)
  n_out = 32 // CW

  def body(p_ref, g_ref, a_ref, w_ref, b_ref, o_ref):
    av = a_ref[...]
    agg = av * (p_ref[0, 0][:, :1] + p_ref[0, 1][:, :1] + g_ref[0][:, :1])
    h = jnp.maximum(agg * w_ref[...] + b_ref[...], 0.0)
    gh = av * h
    for c in range(n_out):
      o_ref[c] = gh[:, c * CW:(c + 1) * CW]

  return pl.pallas_call(
      body, grid=grid,
      in_specs=[pl.BlockSpec((1, NC, TN, W1R), lambda i: (0, 0, i, 0)),
                pl.BlockSpec((1, TN, W1R), lambda i: (0, i, 0)),
                pl.BlockSpec((TN, 1), lambda i: (i, 0)),
                pl.BlockSpec((1, 32), lambda i: (0, 0)),
                pl.BlockSpec((1, 32), lambda i: (0, 0))],
      out_specs=pl.BlockSpec((n_out, TN, CW), lambda i: (0, i, 0)),
      out_shape=jax.ShapeDtypeStruct((n_out, NP, CW), jnp.float32),
  )(P, g1, a, W1, b1)


def _tc_layer(P, g, a, W, b, c_out):
  C = g.shape[0]
  f_out = W.shape[1]
  grid = (NP // TN,)

  def body(p_ref, g_ref, a_ref, w_ref, b_ref, o_ref):
    av = a_ref[...]
    agg = jnp.concatenate(
        [av * (p_ref[c, 0] + p_ref[c, 1] + g_ref[c]) for c in range(C)],
        axis=1)
    h = jnp.maximum(jnp.dot(agg, w_ref[...],
                            preferred_element_type=jnp.float32) +
                    b_ref[...], 0.0)
    gh = av * h
    for c2 in range(c_out):
      o_ref[c2] = gh[:, c2 * CW:(c2 + 1) * CW]

  return pl.pallas_call(
      body, grid=grid,
      in_specs=[pl.BlockSpec((C, NC, TN, CW), lambda i: (0, 0, i, 0)),
                pl.BlockSpec((C, TN, CW), lambda i: (0, i, 0)),
                pl.BlockSpec((TN, 1), lambda i: (i, 0)),
                pl.BlockSpec(W.shape, lambda i: (0, 0)),
                pl.BlockSpec((1, f_out), lambda i: (0, 0))],
      out_specs=pl.BlockSpec((c_out, TN, CW), lambda i: (0, i, 0)),
      out_shape=jax.ShapeDtypeStruct((c_out, NP, CW), jnp.float32),
  )(P, g, a, W, b)


def _tc_layer_pool(P, g, a, W, b, batch3d, fcW1, fcb1, fcW2, fcb2):
  C = g.shape[0]
  f_out = W.shape[1]
  grid = (NP // TN,)
  nb = grid[0]

  def body(p_ref, g_ref, a_ref, w_ref, b_ref, bt_ref,
           w1_ref, c1_ref, w2_ref, c2_ref, o_ref, s_ref, n_ref):
    i = pl.program_id(0)

    @pl.when(i == 0)
    def _():
      s_ref[...] = jnp.zeros_like(s_ref)
      n_ref[...] = jnp.zeros_like(n_ref)

    av = a_ref[...]
    agg = jnp.concatenate(
        [av * (p_ref[c, 0] + p_ref[c, 1] + g_ref[c]) for c in range(C)],
        axis=1)
    h = jnp.maximum(jnp.dot(agg, w_ref[...],
                            preferred_element_type=jnp.float32) +
                    b_ref[...], 0.0)

    ids = bt_ref[0]
    gids = lax.broadcasted_iota(jnp.int32, (G, TN), 0)
    mask = (gids == ids).astype(jnp.float32)
    s_ref[...] += jnp.dot(mask, h, preferred_element_type=jnp.float32)
    n_ref[...] += jnp.sum(mask, axis=1, keepdims=True)

    @pl.when(i == nb - 1)
    def _():
      mean = s_ref[...] / jnp.clip(n_ref[...], 1.0, None)
      r = jnp.maximum(jnp.dot(mean, w1_ref[...],
                              preferred_element_type=jnp.float32) +
                      c1_ref[...], 0.0)
      o_ref[...] = jnp.dot(r, w2_ref[...],
                           preferred_element_type=jnp.float32) + c2_ref[...]

  return pl.pallas_call(
      body, grid=grid,
      in_specs=[pl.BlockSpec((C, NC, TN, CW), lambda i: (0, 0, i, 0)),
                pl.BlockSpec((C, TN, CW), lambda i: (0, i, 0)),
                pl.BlockSpec((TN, 1), lambda i: (i, 0)),
                pl.BlockSpec(W.shape, lambda i: (0, 0)),
                pl.BlockSpec((1, f_out), lambda i: (0, 0)),
                pl.BlockSpec((1, 1, TN), lambda i: (i, 0, 0)),
                pl.BlockSpec((256, 128), lambda i: (0, 0)),
                pl.BlockSpec((1, 128), lambda i: (0, 0)),
                pl.BlockSpec((128, 10), lambda i: (0, 0)),
                pl.BlockSpec((1, 10), lambda i: (0, 0))],
      out_specs=pl.BlockSpec((G, 10), lambda i: (0, 0)),
      out_shape=jax.ShapeDtypeStruct((G, 10), jnp.float32),
      scratch_shapes=[pltpu.VMEM((G, 256), jnp.float32),
                      pltpu.VMEM((G, 1), jnp.float32)],
  )(P, g, a, W, b, batch3d, fcW1, fcb1, fcW2, fcb2)


def kernel(x, edge_index, batch, W1, b1, W2, b2, W3, b3, W4, b4,
           fcW1, fcb1, fcW2, fcb2):
  E = edge_index.shape[1]
  rpt = (E + NW * 128 - 1) // (NW * 128)
  rpt = ((rpt + IDXC - 1) // IDXC) * IDXC
  epr = rpt * NW
  ep = epr * 128
  pad = ep - E
  padv = jnp.full((pad,), N, dtype=jnp.int32)
  src2d = jnp.concatenate([edge_index[0], padv]).reshape(epr, 128)
  dst2d = jnp.concatenate([edge_index[1], padv]).reshape(epr, 128)

  x_pad = jnp.pad(x, ((0, NP - N), (0, 0)))
  batch3d = jnp.pad(batch, (0, NP - N), constant_values=G).reshape(
      NP // TN, 1, TN)

  z1 = jnp.zeros((128, W1R), jnp.float32)
  o1 = jnp.ones((128, W1R), jnp.float32)
  zc = jnp.zeros((128, CW), jnp.float32)

  deg_p = _sc_degree(dst2d, o1, z1)
  a, g1 = _tc_prep(deg_p, x_pad)

  P1 = _sc_agg(g1, src2d, dst2d, z1)
  g2 = _tc_layer1(P1, g1, a, W1.astype(jnp.float32),
                  b1.reshape(1, 32).astype(jnp.float32))

  P2 = _sc_agg(g2, src2d, dst2d, zc)
  g3 = _tc_layer(P2, g2, a, W2, b2.reshape(1, 64), 64 // CW)

  P3 = _sc_agg(g3, src2d, dst2d, zc)
  g4 = _tc_layer(P3, g3, a, W3, b3.reshape(1, 128), 128 // CW)

  P4 = _sc_agg(g4, src2d, dst2d, zc)
  return _tc_layer_pool(P4, g4, a, W4, b4.reshape(1, 256), batch3d,
                        fcW1, fcb1.reshape(1, 128), fcW2, fcb2.reshape(1, 10))
